# baseline (speedup 1.0000x reference)
import functools

import jax
import jax.numpy as jnp
from jax import lax
from jax.experimental import pallas as pl
from jax.experimental.pallas import tpu as pltpu
from jax.experimental.pallas import tpu_sc as plsc

_EMBED_DIM = 64
_NUM_CORES = 2
_NUM_SUBCORES = 16
_NW = _NUM_CORES * _NUM_SUBCORES
_CHUNK = 128
_BLOCK = 512
_GPB = _BLOCK // _CHUNK


@functools.lru_cache(maxsize=None)
def _make_gather(n_rows: int):
    rows_per_w = n_rows // _NW
    n_blocks = rows_per_w // _BLOCK
    chunks_per_w = rows_per_w // _CHUNK
    mesh = plsc.VectorSubcoreMesh(core_axis_name="c", subcore_axis_name="s")

    @functools.partial(
        pl.kernel,
        out_type=jax.ShapeDtypeStruct((n_rows, _EMBED_DIM), jnp.float32),
        mesh=mesh,
        compiler_params=pltpu.CompilerParams(use_tc_tiling_on_sc=False),
        scratch_types=[
            pltpu.VMEM((chunks_per_w, _CHUNK), jnp.int32),
            pltpu.VMEM((_BLOCK, _EMBED_DIM), jnp.float32),
            pltpu.SemaphoreType.DMA,
        ],
    )
    def gather(idx_hbm, table_hbm, out_hbm, idx_v, rows, sem):
        wid = lax.axis_index("s") * _NUM_CORES + lax.axis_index("c")
        base_chunk = wid * chunks_per_w
        base_row = wid * rows_per_w
        pltpu.sync_copy(idx_hbm.at[pl.ds(base_chunk, chunks_per_w)], idx_v)

        @pl.loop(0, n_blocks)
        def _block(b):
            descs = [
                pltpu.async_copy(
                    table_hbm.at[idx_v.at[b * _GPB + j]],
                    rows.at[pl.ds(j * _CHUNK, _CHUNK)],
                    sem,
                )
                for j in range(_GPB)
            ]
            for d in descs:
                d.wait()
            pltpu.sync_copy(
                rows, out_hbm.at[pl.ds(base_row + b * _BLOCK, _BLOCK)]
            )

    return gather


def kernel(inputs, table):
    n = inputs.size
    idx = inputs.reshape(n // _CHUNK, _CHUNK).astype(jnp.int32)
    out = _make_gather(n)(idx, table)
    return out.reshape(inputs.shape + (_EMBED_DIM,))


# speedup vs baseline: 1.0081x; 1.0081x over previous
import functools

import jax
import jax.numpy as jnp
from jax import lax
from jax.experimental import pallas as pl
from jax.experimental.pallas import tpu as pltpu
from jax.experimental.pallas import tpu_sc as plsc

_EMBED_DIM = 64
_NUM_CORES = 2
_NUM_SUBCORES = 16
_NW = _NUM_CORES * _NUM_SUBCORES
_CHUNK = 128
_BLOCK = 256
_GPB = _BLOCK // _CHUNK
_NBUF = 4


@functools.lru_cache(maxsize=None)
def _make_gather(n_rows: int):
    rows_per_w = n_rows // _NW
    blocks_per_w = rows_per_w // _BLOCK
    chunks_per_w = rows_per_w // _CHUNK
    mesh = plsc.VectorSubcoreMesh(core_axis_name="c", subcore_axis_name="s")

    @functools.partial(
        pl.kernel,
        out_type=jax.ShapeDtypeStruct((n_rows, _EMBED_DIM), jnp.float32),
        mesh=mesh,
        compiler_params=pltpu.CompilerParams(use_tc_tiling_on_sc=False),
        scratch_types=[
            pltpu.VMEM((chunks_per_w, _CHUNK), jnp.int32),
            [pltpu.VMEM((_BLOCK, _EMBED_DIM), jnp.float32)] * _NBUF,
            [pltpu.SemaphoreType.DMA] * _NBUF,
            [pltpu.SemaphoreType.DMA] * _NBUF,
        ],
    )
    def gather(idx_hbm, table_hbm, out_hbm, idx_v, bufs, gsems, ssems):
        wid = lax.axis_index("s") * _NUM_CORES + lax.axis_index("c")
        base_chunk = wid * chunks_per_w
        base_row = wid * rows_per_w
        pltpu.sync_copy(idx_hbm.at[pl.ds(base_chunk, chunks_per_w)], idx_v)

        def gather_descs(b, s, start):
            for j in range(_GPB):
                d = pltpu.make_async_copy(
                    table_hbm.at[idx_v.at[b * _GPB + j]],
                    bufs[s].at[pl.ds(j * _CHUNK, _CHUNK)],
                    gsems[s],
                )
                d.start() if start else d.wait()

        def store_desc(b, s, start):
            d = pltpu.make_async_copy(
                bufs[s], out_hbm.at[pl.ds(base_row + b * _BLOCK, _BLOCK)],
                ssems[s],
            )
            d.start() if start else d.wait()

        for s in range(_NBUF):
            gather_descs(s, s, start=True)

        @pl.loop(0, blocks_per_w // _NBUF - 1)
        def _steady(g):
            for s in range(_NBUF):
                b = g * _NBUF + s
                gather_descs(b, s, start=False)
                store_desc(b, s, start=True)
            for s in range(_NBUF):
                b = g * _NBUF + s
                store_desc(b, s, start=False)
                gather_descs(b + _NBUF, s, start=True)

        last = blocks_per_w - _NBUF
        for s in range(_NBUF):
            gather_descs(last + s, s, start=False)
            store_desc(last + s, s, start=True)
        for s in range(_NBUF):
            store_desc(last + s, s, start=False)

    return gather


def kernel(inputs, table):
    n = inputs.size
    idx = inputs.reshape(n // _CHUNK, _CHUNK).astype(jnp.int32)
    out = _make_gather(n)(idx, table)
    return out.reshape(inputs.shape + (_EMBED_DIM,))
